# baseline (speedup 1.0000x reference)
import functools

import jax
import jax.numpy as jnp
from jax import lax
from jax.experimental import pallas as pl
from jax.experimental.pallas import tpu as pltpu
from jax.experimental.pallas import tpu_sc as plsc

NUM_WORKERS = 32
LANES = 16
BATCH = 16384
EMBED_DIM = 64
B_PER_W = BATCH // NUM_WORKERS
CHUNK = 128
N_CHUNKS = B_PER_W // CHUNK


def _transe_body(head_hbm, rel_hbm, ent_hbm, reltab_hbm, out_hbm,
                 hidx, ridx, ent_rows, rel_rows, ent_sem, rel_sem):
    wid = lax.axis_index("s") * 2 + lax.axis_index("c")
    base = wid * B_PER_W

    pltpu.sync_copy(head_hbm.at[pl.ds(base, B_PER_W)], hidx)
    pltpu.sync_copy(rel_hbm.at[pl.ds(base, B_PER_W)], ridx)

    copies = []
    for j in range(N_CHUNKS):
        sl = pl.ds(j * CHUNK, CHUNK)
        copies.append(pltpu.async_copy(
            ent_hbm.at[hidx.at[sl]], ent_rows.at[sl], ent_sem))
        copies.append(pltpu.async_copy(
            reltab_hbm.at[ridx.at[sl]], rel_rows.at[sl], rel_sem))
    for c in copies:
        c.wait()

    def row_add(i, carry):
        for c in range(EMBED_DIM // LANES):
            sl = pl.ds(c * LANES, LANES)
            ent_rows[i, sl] = ent_rows[i, sl] + rel_rows[i, sl]
        return carry

    lax.fori_loop(0, B_PER_W, row_add, 0)

    pltpu.sync_copy(ent_rows, out_hbm.at[pl.ds(base, B_PER_W)])


@jax.jit
def _transe(head, relation, entity_embeddings, relation_embeddings):
    mesh = plsc.VectorSubcoreMesh(core_axis_name="c", subcore_axis_name="s")
    return pl.kernel(
        _transe_body,
        out_type=jax.ShapeDtypeStruct((BATCH, EMBED_DIM), jnp.float32),
        mesh=mesh,
        scratch_types=[
            pltpu.VMEM((B_PER_W,), jnp.int32),
            pltpu.VMEM((B_PER_W,), jnp.int32),
            pltpu.VMEM((B_PER_W, EMBED_DIM), jnp.float32),
            pltpu.VMEM((B_PER_W, EMBED_DIM), jnp.float32),
            pltpu.SemaphoreType.DMA,
            pltpu.SemaphoreType.DMA,
        ],
        compiler_params=pltpu.CompilerParams(use_tc_tiling_on_sc=False),
    )(head, relation, entity_embeddings, relation_embeddings)


def kernel(head, relation, entity_embeddings, relation_embeddings):
    return _transe(head, relation, entity_embeddings, relation_embeddings)
